# Initial kernel scaffold; baseline (speedup 1.0000x reference)
import functools

import jax
import jax.numpy as jnp
from jax import lax
from jax.experimental import pallas as pl
from jax.experimental.pallas import tpu as pltpu
from jax.experimental.pallas import tpu_sc as plsc

_VOCAB = 1000000
_D = 32
_BATCH = 16384
_HIST = 50
_TOT = _BATCH * _HIST
_NC, _NS = 2, 16
_NW = _NC * _NS
_PER_W = _TOT // _NW
_CHUNK = 128
_NCH = _PER_W // _CHUNK

_mesh = plsc.VectorSubcoreMesh(core_axis_name="c", subcore_axis_name="s")


@functools.partial(
    pl.kernel,
    out_type=jax.ShapeDtypeStruct((_TOT, _D), jnp.float32),
    mesh=_mesh,
    scratch_types=[
        pltpu.VMEM((_NCH, _CHUNK), jnp.int32),
        pltpu.VMEM((_CHUNK, _D), jnp.float32),
        pltpu.SemaphoreType.DMA,
    ],
)
def _sc_gather(idx_hbm, table_hbm, out_hbm, idx_v, rows_v, sem):
    wid = lax.axis_index("s") * _NC + lax.axis_index("c")
    base = wid * _PER_W
    pltpu.sync_copy(idx_hbm.at[wid], idx_v)

    def body(j, carry):
        pltpu.async_copy(table_hbm.at[idx_v.at[j]], rows_v, sem).wait()
        pltpu.sync_copy(rows_v, out_hbm.at[pl.ds(base + j * _CHUNK, _CHUNK)])
        return carry

    lax.fori_loop(0, _NCH, body, 0)


def kernel(input_ids, table):
    idx = input_ids.astype(jnp.int32).reshape(_NW, _NCH, _CHUNK)
    out = _sc_gather(idx, table)
    return out.reshape(_BATCH, _HIST, _D)


# speedup vs baseline: 1.0228x; 1.0228x over previous
import functools

import jax
import jax.numpy as jnp
from jax import lax
from jax.experimental import pallas as pl
from jax.experimental.pallas import tpu as pltpu
from jax.experimental.pallas import tpu_sc as plsc

_VOCAB = 1000000
_D = 32
_BATCH = 16384
_HIST = 50
_TOT = _BATCH * _HIST
_NC, _NS = 2, 16
_NW = _NC * _NS
_PER_W = _TOT // _NW
_CHUNK = 128
_NCH = _PER_W // _CHUNK

_mesh = plsc.VectorSubcoreMesh(core_axis_name="c", subcore_axis_name="s")


@functools.partial(
    pl.kernel,
    out_type=jax.ShapeDtypeStruct((_TOT, _D), jnp.float32),
    mesh=_mesh,
    scratch_types=[
        pltpu.VMEM((_NCH, _CHUNK), jnp.int32),
        pltpu.VMEM((_CHUNK, _D), jnp.float32),
        pltpu.SemaphoreType.DMA,
    ],
    compiler_params=pltpu.CompilerParams(use_tc_tiling_on_sc=False),
)
def _sc_gather(idx_hbm, table_hbm, out_hbm, idx_v, rows_v, sem):
    wid = lax.axis_index("s") * _NC + lax.axis_index("c")
    base = wid * _PER_W
    pltpu.sync_copy(idx_hbm.at[wid], idx_v)

    def body(j, carry):
        pltpu.async_copy(table_hbm.at[idx_v.at[j]], rows_v, sem).wait()
        pltpu.sync_copy(rows_v, out_hbm.at[pl.ds(base + j * _CHUNK, _CHUNK)])
        return carry

    lax.fori_loop(0, _NCH, body, 0)


def kernel(input_ids, table):
    idx = input_ids.astype(jnp.int32).reshape(_NW, _NCH, _CHUNK)
    out = _sc_gather(idx, table)
    return out.reshape(_BATCH, _HIST, _D)


# speedup vs baseline: 1.3108x; 1.2817x over previous
import functools

import jax
import jax.numpy as jnp
from jax import lax
from jax.experimental import pallas as pl
from jax.experimental.pallas import tpu as pltpu
from jax.experimental.pallas import tpu_sc as plsc

_VOCAB = 1000000
_D = 32
_BATCH = 16384
_HIST = 50
_TOT = _BATCH * _HIST
_NC, _NS = 2, 16
_NW = _NC * _NS
_PER_W = _TOT // _NW
_CHUNK = 128
_NCH = _PER_W // _CHUNK
_NBUF = 16

_mesh = plsc.VectorSubcoreMesh(core_axis_name="c", subcore_axis_name="s")


@functools.partial(
    pl.kernel,
    out_type=jax.ShapeDtypeStruct((_TOT // _CHUNK, _CHUNK, _D), jnp.float32),
    mesh=_mesh,
    scratch_types=[
        pltpu.VMEM((_NCH, _CHUNK), jnp.int32),
        pltpu.VMEM((_NBUF, _CHUNK, _D), jnp.float32),
        pltpu.SemaphoreType.DMA,
        pltpu.SemaphoreType.DMA,
    ],
    compiler_params=pltpu.CompilerParams(use_tc_tiling_on_sc=False),
)
def _sc_gather(idx_hbm, table_hbm, out_hbm, idx_v, buf, gsem, ssem):
    wid = lax.axis_index("s") * _NC + lax.axis_index("c")
    obase = wid * _NCH
    pltpu.sync_copy(idx_hbm.at[wid], idx_v)

    for j in range(_NBUF - 1):
        pltpu.async_copy(table_hbm.at[idx_v.at[j]], buf.at[j], gsem)

    def body(j, carry):
        b = lax.rem(j, _NBUF)
        pltpu.make_async_copy(table_hbm.at[idx_v.at[j]], buf.at[b], gsem).wait()
        pltpu.async_copy(buf.at[b], out_hbm.at[obase + j], ssem)

        @pl.when(j + _NBUF - 1 < _NCH)
        def _start_next():
            @pl.when(j >= 1)
            def _drain_store():
                pltpu.make_async_copy(
                    buf.at[0], out_hbm.at[obase], ssem).wait()
            pltpu.async_copy(
                table_hbm.at[idx_v.at[j + _NBUF - 1]],
                buf.at[lax.rem(j + _NBUF - 1, _NBUF)], gsem)

        return carry

    lax.fori_loop(0, _NCH, body, 0)

    for _ in range(_NBUF):
        pltpu.make_async_copy(buf.at[0], out_hbm.at[obase], ssem).wait()


def kernel(input_ids, table):
    idx = input_ids.astype(jnp.int32).reshape(_NW, _NCH, _CHUNK)
    out = _sc_gather(idx, table)
    return out.reshape(_BATCH, _HIST, _D)


# speedup vs baseline: 1.9450x; 1.4838x over previous
import functools

import jax
import jax.numpy as jnp
from jax import lax
from jax.experimental import pallas as pl
from jax.experimental.pallas import tpu as pltpu
from jax.experimental.pallas import tpu_sc as plsc

_VOCAB = 1000000
_D = 32
_BATCH = 16384
_HIST = 50
_NC, _NS = 2, 16
_NW = _NC * _NS
_BW = _BATCH // _NW
_CHUNK = 128
_KPH = _BW // _CHUNK
_NCH = _HIST * _KPH
_NBUF = 16

_mesh = plsc.VectorSubcoreMesh(core_axis_name="c", subcore_axis_name="s")


@functools.partial(
    pl.kernel,
    out_type=jax.ShapeDtypeStruct((_HIST, _BATCH, _D), jnp.float32),
    mesh=_mesh,
    scratch_types=[
        pltpu.VMEM((_HIST, _BW), jnp.int32),
        pltpu.VMEM((_NBUF, _CHUNK, _D), jnp.float32),
        pltpu.SemaphoreType.DMA,
        pltpu.SemaphoreType.DMA,
    ],
    compiler_params=pltpu.CompilerParams(use_tc_tiling_on_sc=False),
)
def _sc_gather(idx_hbm, table_hbm, out_hbm, idx_v, buf, gsem, ssem):
    wid = lax.axis_index("s") * _NC + lax.axis_index("c")
    b0 = wid * _BW
    pltpu.sync_copy(idx_hbm.at[:, pl.ds(b0, _BW)], idx_v)

    def chunk_refs(j):
        h = lax.div(j, _KPH)
        k = lax.rem(j, _KPH)
        src = table_hbm.at[idx_v.at[h, pl.ds(k * _CHUNK, _CHUNK)]]
        dst = out_hbm.at[h, pl.ds(b0 + k * _CHUNK, _CHUNK)]
        return src, dst

    for j in range(_NBUF - 1):
        src, _ = chunk_refs(j)
        pltpu.async_copy(src, buf.at[j], gsem)

    def body(j, carry):
        b = lax.rem(j, _NBUF)
        src, dst = chunk_refs(j)
        pltpu.make_async_copy(src, buf.at[b], gsem).wait()
        pltpu.async_copy(buf.at[b], dst, ssem)

        @pl.when(j + _NBUF - 1 < _NCH)
        def _start_next():
            @pl.when(j >= 1)
            def _drain_store():
                pltpu.make_async_copy(
                    buf.at[0], out_hbm.at[0, pl.ds(b0, _CHUNK)], ssem).wait()
            nsrc, _ = chunk_refs(j + _NBUF - 1)
            pltpu.async_copy(nsrc, buf.at[lax.rem(j + _NBUF - 1, _NBUF)], gsem)

        return carry

    lax.fori_loop(0, _NCH, body, 0)

    for _ in range(_NBUF):
        pltpu.make_async_copy(
            buf.at[0], out_hbm.at[0, pl.ds(b0, _CHUNK)], ssem).wait()


def kernel(input_ids, table):
    idx_t = input_ids.astype(jnp.int32).T
    out = _sc_gather(idx_t, table)
    return out.transpose(1, 0, 2)
